# baseline (speedup 1.0000x reference)
import functools

import jax
import jax.numpy as jnp
from jax import lax
from jax.experimental import pallas as pl
from jax.experimental.pallas import tpu as pltpu
from jax.experimental.pallas import tpu_sc as plsc

N_FACTORS = 64
BATCH = 16384
NW = 32
B_PER_W = BATCH // NW
N_CHUNK = 4
CHUNK = B_PER_W // N_CHUNK


def _body(users_hbm, items_hbm, ut_hbm, it_hbm, out_hbm,
          uidx, iidx, urows, irows, out_v, sem):
    wid = lax.axis_index("s") * 2 + lax.axis_index("c")

    pltpu.sync_copy(users_hbm.at[wid], uidx)
    pltpu.sync_copy(items_hbm.at[wid], iidx)

    copies = []
    for j in range(N_CHUNK):
        copies.append(pltpu.async_copy(
            ut_hbm.at[uidx.at[j]], urows.at[pl.ds(j * CHUNK, CHUNK)], sem))
        copies.append(pltpu.async_copy(
            it_hbm.at[iidx.at[j]], irows.at[pl.ds(j * CHUNK, CHUNK)], sem))
    for cp in copies:
        cp.wait()

    iota = lax.iota(jnp.int32, 16)

    def group(g, _):
        rvec = g * 16 + iota
        acc = jnp.zeros((16,), jnp.float32)
        for d in range(N_FACTORS):
            colv = (iota + d) & (N_FACTORS - 1)
            u = plsc.load_gather(urows, [rvec, colv])
            v = plsc.load_gather(irows, [rvec, colv])
            acc = acc + u * v
        out_v[pl.ds(g * 16, 16)] = acc
        return 0

    lax.fori_loop(0, B_PER_W // 16, group, 0)

    pltpu.sync_copy(out_v, out_hbm.at[wid])


@functools.partial(jax.jit, static_argnames=())
def _mf(users2, items2, user_table, item_table):
    mesh = plsc.VectorSubcoreMesh(core_axis_name="c", subcore_axis_name="s")
    f = pl.kernel(
        _body,
        out_type=jax.ShapeDtypeStruct((NW, B_PER_W), jnp.float32),
        mesh=mesh,
        scratch_types=[
            pltpu.VMEM((N_CHUNK, CHUNK), jnp.int32),
            pltpu.VMEM((N_CHUNK, CHUNK), jnp.int32),
            pltpu.VMEM((B_PER_W, N_FACTORS), jnp.float32),
            pltpu.VMEM((B_PER_W, N_FACTORS), jnp.float32),
            pltpu.VMEM((B_PER_W,), jnp.float32),
            pltpu.SemaphoreType.DMA,
        ],
        compiler_params=pltpu.CompilerParams(
            needs_layout_passes=False, use_tc_tiling_on_sc=False),
    )
    return f(users2, items2, user_table, item_table)


def kernel(users, items, user_table, item_table):
    users2 = users.reshape(NW, N_CHUNK, CHUNK)
    items2 = items.reshape(NW, N_CHUNK, CHUNK)
    out = _mf(users2, items2, user_table, item_table)
    return out.reshape(BATCH)


# speedup vs baseline: 1.5633x; 1.5633x over previous
import jax
import jax.numpy as jnp
from jax import lax
from jax.experimental import pallas as pl
from jax.experimental.pallas import tpu as pltpu
from jax.experimental.pallas import tpu_sc as plsc

N_FACTORS = 64
BATCH = 16384
NW = 32
B_PER_W = BATCH // NW
HALF = B_PER_W // 2
ROW_BYTES = N_FACTORS * 4


def _body(users_hbm, items_hbm, ut_hbm, it_hbm, out_hbm,
          uidx, iidx, urows, irows, out_v, sem):
    wid = lax.axis_index("s") * 2 + lax.axis_index("c")

    pltpu.sync_copy(users_hbm.at[wid], uidx.at[pl.ds(0, B_PER_W)])
    pltpu.sync_copy(items_hbm.at[wid], iidx.at[pl.ds(0, B_PER_W)])

    iota = lax.iota(jnp.int32, 16)

    for half in range(2):
        def fire(t, _, half=half):
            t2 = half * (HALF // 8) + t
            uvec = uidx[pl.ds(t2 * 8, 16)]
            ivec = iidx[pl.ds(t2 * 8, 16)]
            for l in range(8):
                slot = t * 8 + l
                pltpu.async_copy(ut_hbm.at[uvec[l]], urows.at[slot], sem)
                pltpu.async_copy(it_hbm.at[ivec[l]], irows.at[slot], sem)
            return 0

        lax.fori_loop(0, HALF // 8, fire, 0)

        pltpu.make_async_copy(ut_hbm.at[pl.ds(0, HALF)], urows, sem).wait()
        pltpu.make_async_copy(it_hbm.at[pl.ds(0, HALF)], irows, sem).wait()

        def group(g, _, half=half):
            rowv = g * 16 + iota
            acc = jnp.zeros((16,), jnp.float32)
            for d in range(N_FACTORS):
                colv = (iota + d) & (N_FACTORS - 1)
                u = plsc.load_gather(urows, [rowv, colv])
                v = plsc.load_gather(irows, [rowv, colv])
                acc = acc + u * v
            out_v[pl.ds(half * HALF + g * 16, 16)] = acc
            return 0

        lax.fori_loop(0, HALF // 16, group, 0)

    pltpu.sync_copy(out_v, out_hbm.at[wid])


@jax.jit
def _mf(users2, items2, user_table, item_table):
    mesh = plsc.VectorSubcoreMesh(core_axis_name="c", subcore_axis_name="s")
    f = pl.kernel(
        _body,
        out_type=jax.ShapeDtypeStruct((NW, B_PER_W), jnp.float32),
        mesh=mesh,
        scratch_types=[
            pltpu.VMEM((B_PER_W + 16,), jnp.int32),
            pltpu.VMEM((B_PER_W + 16,), jnp.int32),
            pltpu.VMEM((HALF, N_FACTORS), jnp.float32),
            pltpu.VMEM((HALF, N_FACTORS), jnp.float32),
            pltpu.VMEM((B_PER_W,), jnp.float32),
            pltpu.SemaphoreType.DMA,
        ],
        compiler_params=pltpu.CompilerParams(needs_layout_passes=False),
    )
    return f(users2, items2, user_table, item_table)


def kernel(users, items, user_table, item_table):
    users2 = users.reshape(NW, B_PER_W)
    items2 = items.reshape(NW, B_PER_W)
    out = _mf(users2, items2, user_table, item_table)
    return out.reshape(BATCH)
